# baseline (speedup 1.0000x reference)
import functools

import jax
import jax.numpy as jnp
from jax import lax
from jax.experimental import pallas as pl
from jax.experimental.pallas import tpu as pltpu
from jax.experimental.pallas import tpu_sc as plsc

_MESH = plsc.VectorSubcoreMesh(core_axis_name="c", subcore_axis_name="s")


@functools.partial(
    pl.kernel,
    out_type=jax.ShapeDtypeStruct((4,), jnp.float32),
    mesh=_MESH,
    scratch_types=[
        pltpu.VMEM((16,), jnp.float32),
        pltpu.VMEM((16,), jnp.float32),
    ],
    compiler_params=pltpu.CompilerParams(needs_layout_passes=False),
)
def _gather_sc(x_hbm, out_hbm, buf_v, res_v):
    cid = lax.axis_index("c")
    sid = lax.axis_index("s")

    @pl.when(jnp.logical_and(cid == 0, sid == 0))
    def _():
        pltpu.sync_copy(x_hbm.at[pl.ds(0, 16)], buf_v)
        lane = lax.iota(jnp.int32, 16)
        idx = jnp.where(
            lane == 0,
            2,
            jnp.where(lane == 1, 0, jnp.where(lane == 2, 4, jnp.where(lane == 3, 1, 0))),
        )
        res_v[...] = plsc.load_gather(buf_v, [idx])
        pltpu.sync_copy(res_v.at[pl.ds(0, 4)], out_hbm)


def kernel(x):
    return _gather_sc(x)


# speedup vs baseline: 1.0761x; 1.0761x over previous
import functools

import jax
import jax.numpy as jnp
from jax import lax
from jax.experimental import pallas as pl
from jax.experimental.pallas import tpu as pltpu
from jax.experimental.pallas import tpu_sc as plsc

_MESH = plsc.VectorSubcoreMesh(
    core_axis_name="c", subcore_axis_name="s", num_cores=1, num_subcores=1
)


@functools.partial(
    pl.kernel,
    out_type=jax.ShapeDtypeStruct((4,), jnp.float32),
    mesh=_MESH,
    scratch_types=[
        pltpu.VMEM((16,), jnp.float32),
        pltpu.VMEM((16,), jnp.float32),
    ],
    compiler_params=pltpu.CompilerParams(needs_layout_passes=False),
)
def _gather_sc(x_hbm, out_hbm, buf_v, res_v):
    cid = lax.axis_index("c")
    sid = lax.axis_index("s")

    @pl.when(jnp.logical_and(cid == 0, sid == 0))
    def _():
        pltpu.sync_copy(x_hbm.at[pl.ds(0, 16)], buf_v)
        lane = lax.iota(jnp.int32, 16)
        idx = jnp.where(
            lane == 0,
            2,
            jnp.where(lane == 1, 0, jnp.where(lane == 2, 4, jnp.where(lane == 3, 1, 0))),
        )
        res_v[...] = plsc.load_gather(buf_v, [idx])
        pltpu.sync_copy(res_v.at[pl.ds(0, 4)], out_hbm)


def kernel(x):
    return _gather_sc(x)


# speedup vs baseline: 1.0796x; 1.0033x over previous
import functools

import jax
import jax.numpy as jnp
from jax import lax
from jax.experimental import pallas as pl
from jax.experimental.pallas import tpu as pltpu
from jax.experimental.pallas import tpu_sc as plsc

_MESH = plsc.VectorSubcoreMesh(
    core_axis_name="c", subcore_axis_name="s", num_cores=1, num_subcores=1
)


@functools.partial(
    pl.kernel,
    out_type=jax.ShapeDtypeStruct((4,), jnp.float32),
    mesh=_MESH,
    scratch_types=[
        pltpu.VMEM((16,), jnp.float32),
        pltpu.VMEM((16,), jnp.float32),
    ],
    compiler_params=pltpu.CompilerParams(
        needs_layout_passes=False, skip_device_barrier=True
    ),
)
def _gather_sc(x_hbm, out_hbm, buf_v, res_v):
    cid = lax.axis_index("c")
    sid = lax.axis_index("s")

    @pl.when(jnp.logical_and(cid == 0, sid == 0))
    def _():
        pltpu.sync_copy(x_hbm.at[pl.ds(0, 16)], buf_v)
        lane = lax.iota(jnp.int32, 16)
        idx = jnp.where(
            lane == 0,
            2,
            jnp.where(lane == 1, 0, jnp.where(lane == 2, 4, jnp.where(lane == 3, 1, 0))),
        )
        res_v[...] = plsc.load_gather(buf_v, [idx])
        pltpu.sync_copy(res_v.at[pl.ds(0, 4)], out_hbm)


def kernel(x):
    return _gather_sc(x)


# speedup vs baseline: 12.9700x; 12.0133x over previous
import jax
import jax.numpy as jnp
from jax import lax
from jax.experimental import pallas as pl


def _body(x_ref, o_ref):
    vals = x_ref[...]
    col = lax.broadcasted_iota(jnp.int32, (4, 128), 1)
    row = lax.broadcasted_iota(jnp.int32, (4, 128), 0)
    sel = jnp.where(row == 0, 2, jnp.where(row == 1, 0, jnp.where(row == 2, 4, 1)))
    onehot = jnp.where(col == sel, vals[None, :], 0.0)
    o_ref[...] = jnp.sum(onehot, axis=1)


def kernel(x):
    return pl.pallas_call(
        _body,
        out_shape=jax.ShapeDtypeStruct((4,), jnp.float32),
        grid=(1,),
        in_specs=[pl.BlockSpec((128,), lambda i: (0,))],
        out_specs=pl.BlockSpec((4,), lambda i: (0,)),
    )(x)
